# baseline (speedup 1.0000x reference)
import jax
import jax.numpy as jnp
from jax import lax
from jax.experimental import pallas as pl
from jax.experimental.pallas import tpu as pltpu
from jax.experimental.pallas import tpu_sc as plsc

N = 10000
D = 128
E = 320000

NC = 2
NS = 16
NT = NC * NS

NP = 10240
NZ = 2048
NSC = N + NZ
Z = N
NPT = NP // NS

CH = 128
GP = 40
G = 80
NPH = G // GP
TOTR = NT * G
EPAD = TOTR * CH

CHE = 64
GE = 160
GPE = 40
NPHE = GE // GPE
RE = 4

_MESH = plsc.VectorSubcoreMesh(
    core_axis_name="c", subcore_axis_name="s", num_cores=NC, num_subcores=NS
)


def _deg_body(src_hbm, dst_hbm, degp_out, dfix_out,
              src_all, dst_all, dfix_all, val_all, zer_v, deg_sh, sem):
    cid = lax.axis_index("c")
    sid = lax.axis_index("s")
    wid = cid * NS + sid
    base_row = wid * G

    def _fill_zeros(i, carry):
        zer_v[pl.ds(i * 16, 16)] = jnp.zeros((16,), jnp.float32)
        return carry

    lax.fori_loop(0, NPT // 16, _fill_zeros, 0)
    pltpu.sync_copy(zer_v, deg_sh.at[pl.ds(sid * NPT, NPT)])
    plsc.subcore_barrier()

    def _phase(p, carry):
        row = base_row + p * GP
        pltpu.sync_copy(src_hbm.at[pl.ds(row, GP)], src_all)
        pltpu.sync_copy(dst_hbm.at[pl.ds(row, GP)], dst_all)

        def _fix_row(g, c1):
            def _fix(j, c2):
                s = src_all[g, pl.ds(j * 16, 16)]
                d = dst_all[g, pl.ds(j * 16, 16)]
                diag = s == d
                lane = lax.iota(jnp.int32, 16)
                zidx = Z + (((row + g) * CH + j * 16 + lane) & (NZ - 1))
                dfix_all[g, pl.ds(j * 16, 16)] = jnp.where(diag, zidx, d)
                val_all[g, pl.ds(j * 16, 16)] = jnp.where(
                    diag, jnp.zeros((16,), jnp.float32),
                    jnp.ones((16,), jnp.float32))
                return c2

            lax.fori_loop(0, CH // 16, _fix, 0)
            return c1

        lax.fori_loop(0, GP, _fix_row, 0)
        pltpu.sync_copy(dfix_all, dfix_out.at[pl.ds(row, GP)])

        def _wave(kk, c1):
            for r in range(8):
                g = kk * 8 + r
                pltpu.async_copy(val_all.at[g], deg_sh.at[src_all.at[g]],
                                 sem, add=True)
            for r in range(8):
                pltpu.make_async_copy(val_all.at[0],
                                      deg_sh.at[src_all.at[0]], sem).wait()
            return c1

        lax.fori_loop(0, GP // 8, _wave, 0)
        return carry

    lax.fori_loop(0, NPH, _phase, 0)
    plsc.subcore_barrier()
    pltpu.sync_copy(deg_sh.at[pl.ds(sid * NPT, NPT)], zer_v)
    pltpu.sync_copy(zer_v, degp_out.at[cid, pl.ds(sid * NPT, NPT)])


_deg = pl.kernel(
    _deg_body,
    out_type=(
        jax.ShapeDtypeStruct((NC, NP), jnp.float32),
        jax.ShapeDtypeStruct((TOTR, CH), jnp.int32),
    ),
    mesh=_MESH,
    scratch_types=[
        pltpu.VMEM((GP, CH), jnp.int32),
        pltpu.VMEM((GP, CH), jnp.int32),
        pltpu.VMEM((GP, CH), jnp.int32),
        pltpu.VMEM((GP, CH), jnp.float32),
        pltpu.VMEM((NPT,), jnp.float32),
        pltpu.VMEM_SHARED((NP,), jnp.float32),
        pltpu.SemaphoreType.DMA,
    ],
)


def _edge_body(scaled_hbm, dfix_hbm, src_hbm, agg_out,
               dfix_h, src_h, rows0, rows1, rows2, rows3, acc_sh,
               g0, g1, g2, g3, s0, s1, s2, s3):
    cid = lax.axis_index("c")
    sid = lax.axis_index("s")
    wid = cid * NS + sid
    base_row = wid * GE
    rows = (rows0, rows1, rows2, rows3)
    gs = (g0, g1, g2, g3)
    ss = (s0, s1, s2, s3)

    def _zrow(i, carry):
        def _zcol(j, c2):
            rows0[i, pl.ds(j * 16, 16)] = jnp.zeros((16,), jnp.float32)
            return c2

        lax.fori_loop(0, D // 16, _zcol, 0)
        return carry

    lax.fori_loop(0, CHE, _zrow, 0)

    def _zacc(k, carry):
        pltpu.sync_copy(rows0, acc_sh.at[pl.ds(sid * NPT + k * CHE, CHE)])
        return carry

    lax.fori_loop(0, NPT // CHE, _zacc, 0)
    plsc.subcore_barrier()

    def _w_gather(r):
        pltpu.make_async_copy(scaled_hbm.at[dfix_h.at[0]], rows[r],
                              gs[r]).wait()

    def _w_scatter(r):
        pltpu.make_async_copy(rows[r], acc_sh.at[src_h.at[0]], ss[r]).wait()

    def _phase(p, carry):
        row = base_row + p * GPE
        pltpu.sync_copy(dfix_hbm.at[pl.ds(row, GPE)], dfix_h)
        pltpu.sync_copy(src_hbm.at[pl.ds(row, GPE)], src_h)

        for r in range(RE):
            pltpu.async_copy(scaled_hbm.at[dfix_h.at[r]], rows[r], gs[r])
            if r >= 1:
                _w_gather(r - 1)
                pltpu.async_copy(rows[r - 1], acc_sh.at[src_h.at[r - 1]],
                                 ss[r - 1], add=True)

        def _steady(kk, c1):
            for r in range(RE):
                g = kk * RE + r
                _w_scatter(r)
                pltpu.async_copy(scaled_hbm.at[dfix_h.at[g]], rows[r], gs[r])
                q = (r - 1) % RE
                _w_gather(q)
                pltpu.async_copy(rows[q], acc_sh.at[src_h.at[g - 1]],
                                 ss[q], add=True)
            return c1

        lax.fori_loop(1, GPE // RE, _steady, 0)

        _w_gather(RE - 1)
        pltpu.async_copy(rows[RE - 1], acc_sh.at[src_h.at[GPE - 1]],
                         ss[RE - 1], add=True)
        for r in range(RE):
            _w_scatter(r)
        return carry

    lax.fori_loop(0, NPHE, _phase, 0)
    plsc.subcore_barrier()

    def _dump(k, carry):
        sl = pl.ds(sid * NPT + k * CHE, CHE)
        pltpu.sync_copy(acc_sh.at[sl], rows0)
        pltpu.sync_copy(rows0, agg_out.at[cid, sl])
        return carry

    lax.fori_loop(0, NPT // CHE, _dump, 0)


_edge = pl.kernel(
    _edge_body,
    out_type=jax.ShapeDtypeStruct((NC, NP, D), jnp.float32),
    mesh=_MESH,
    scratch_types=[
        pltpu.VMEM((GPE, CHE), jnp.int32),
        pltpu.VMEM((GPE, CHE), jnp.int32),
        pltpu.VMEM((CHE, D), jnp.float32),
        pltpu.VMEM((CHE, D), jnp.float32),
        pltpu.VMEM((CHE, D), jnp.float32),
        pltpu.VMEM((CHE, D), jnp.float32),
        pltpu.VMEM_SHARED((NP, D), jnp.float32),
        pltpu.SemaphoreType.DMA,
        pltpu.SemaphoreType.DMA,
        pltpu.SemaphoreType.DMA,
        pltpu.SemaphoreType.DMA,
        pltpu.SemaphoreType.DMA,
        pltpu.SemaphoreType.DMA,
        pltpu.SemaphoreType.DMA,
        pltpu.SemaphoreType.DMA,
    ],
)

BM = 1024


def _mm_body(x_ref, w_ref, d0_ref, d1_ref, scaled_ref, dinv_ref):
    deg = 1.0 + d0_ref[...] + d1_ref[...]
    dinv = lax.rsqrt(deg)
    acc = jnp.dot(x_ref[...], w_ref[...], preferred_element_type=jnp.float32)
    scaled_ref[...] = acc * dinv[:, None]
    dinv_ref[...] = dinv


_mm = pl.pallas_call(
    _mm_body,
    grid=(pl.cdiv(NSC, BM),),
    in_specs=[
        pl.BlockSpec((BM, D), lambda i: (i, 0)),
        pl.BlockSpec((D, D), lambda i: (0, 0)),
        pl.BlockSpec((BM,), lambda i: (i,)),
        pl.BlockSpec((BM,), lambda i: (i,)),
    ],
    out_specs=[
        pl.BlockSpec((BM, D), lambda i: (i, 0)),
        pl.BlockSpec((BM,), lambda i: (i,)),
    ],
    out_shape=[
        jax.ShapeDtypeStruct((NSC, D), jnp.float32),
        jax.ShapeDtypeStruct((NSC,), jnp.float32),
    ],
)


def _comb_body(agg_ref0, agg_ref1, s_ref, dinv_ref, b_ref, o_ref):
    total = agg_ref0[0] + agg_ref1[0] + s_ref[...]
    o_ref[...] = total * dinv_ref[...][:, None] + b_ref[...]


_comb = pl.pallas_call(
    _comb_body,
    grid=(pl.cdiv(N, BM),),
    in_specs=[
        pl.BlockSpec((1, BM, D), lambda i: (0, i, 0)),
        pl.BlockSpec((1, BM, D), lambda i: (1, i, 0)),
        pl.BlockSpec((BM, D), lambda i: (i, 0)),
        pl.BlockSpec((BM,), lambda i: (i,)),
        pl.BlockSpec((1, D), lambda i: (0, 0)),
    ],
    out_specs=pl.BlockSpec((BM, D), lambda i: (i, 0)),
    out_shape=jax.ShapeDtypeStruct((N, D), jnp.float32),
)


def kernel(input, adj, weight, bias):
    src = adj[0]
    dst = adj[1]
    pad = EPAD - E
    padv = jnp.arange(pad, dtype=jnp.int32)
    src2d = jnp.concatenate([src, padv]).reshape(TOTR, CH)
    dst2d = jnp.concatenate([dst, padv]).reshape(TOTR, CH)
    x_p = jnp.pad(input, ((0, NSC - N), (0, 0)))

    degp, dfix2d = _deg(src2d, dst2d)
    d0 = jnp.pad(degp[0], (0, NSC - NP))
    d1 = jnp.pad(degp[1], (0, NSC - NP))
    scaled, dinv = _mm(x_p, weight, d0, d1)
    aggp = _edge(scaled, dfix2d.reshape(NT * GE, CHE),
                 src2d.reshape(NT * GE, CHE))
    return _comb(aggp, aggp, scaled, dinv, bias.reshape(1, D))


# speedup vs baseline: 1.1155x; 1.1155x over previous
import jax
import jax.numpy as jnp
from jax import lax
from jax.experimental import pallas as pl
from jax.experimental.pallas import tpu as pltpu
from jax.experimental.pallas import tpu_sc as plsc

N = 10000
D = 128
E = 320000

NC = 2
NS = 16
NT = NC * NS

NP = 10240
NZ = 2048
NSC = N + NZ
Z = N
NPT = NP // NS

CH = 128
GP = 40
G = 80
NPH = G // GP
TOTR = NT * G
EPAD = TOTR * CH

CHE = 128
GE = 80
GPE = 40
NPHE = GE // GPE
RE = 2

_MESH = plsc.VectorSubcoreMesh(
    core_axis_name="c", subcore_axis_name="s", num_cores=NC, num_subcores=NS
)


def _deg_body(src_hbm, dst_hbm, degp_out, dfix_out,
              src_all, dst_all, dfix_all, val_all, zer_v, deg_sh, sem):
    cid = lax.axis_index("c")
    sid = lax.axis_index("s")
    wid = cid * NS + sid
    base_row = wid * G

    def _fill_zeros(i, carry):
        zer_v[pl.ds(i * 16, 16)] = jnp.zeros((16,), jnp.float32)
        return carry

    lax.fori_loop(0, NPT // 16, _fill_zeros, 0)
    pltpu.sync_copy(zer_v, deg_sh.at[pl.ds(sid * NPT, NPT)])
    plsc.subcore_barrier()

    def _phase(p, carry):
        row = base_row + p * GP
        pltpu.sync_copy(src_hbm.at[pl.ds(row, GP)], src_all)
        pltpu.sync_copy(dst_hbm.at[pl.ds(row, GP)], dst_all)

        def _fix_row(g, c1):
            def _fix(j, c2):
                s = src_all[g, pl.ds(j * 16, 16)]
                d = dst_all[g, pl.ds(j * 16, 16)]
                diag = s == d
                lane = lax.iota(jnp.int32, 16)
                zidx = Z + (((row + g) * CH + j * 16 + lane) & (NZ - 1))
                dfix_all[g, pl.ds(j * 16, 16)] = jnp.where(diag, zidx, d)
                val_all[g, pl.ds(j * 16, 16)] = jnp.where(
                    diag, jnp.zeros((16,), jnp.float32),
                    jnp.ones((16,), jnp.float32))
                return c2

            lax.fori_loop(0, CH // 16, _fix, 0)
            return c1

        lax.fori_loop(0, GP, _fix_row, 0)
        pltpu.sync_copy(dfix_all, dfix_out.at[pl.ds(row, GP)])

        def _wave(kk, c1):
            for r in range(8):
                g = kk * 8 + r
                pltpu.async_copy(val_all.at[g], deg_sh.at[src_all.at[g]],
                                 sem, add=True)
            for r in range(8):
                pltpu.make_async_copy(val_all.at[0],
                                      deg_sh.at[src_all.at[0]], sem).wait()
            return c1

        lax.fori_loop(0, GP // 8, _wave, 0)
        return carry

    lax.fori_loop(0, NPH, _phase, 0)
    plsc.subcore_barrier()
    pltpu.sync_copy(deg_sh.at[pl.ds(sid * NPT, NPT)], zer_v)
    pltpu.sync_copy(zer_v, degp_out.at[cid, pl.ds(sid * NPT, NPT)])


_deg = pl.kernel(
    _deg_body,
    out_type=(
        jax.ShapeDtypeStruct((NC, NP), jnp.float32),
        jax.ShapeDtypeStruct((TOTR, CH), jnp.int32),
    ),
    mesh=_MESH,
    scratch_types=[
        pltpu.VMEM((GP, CH), jnp.int32),
        pltpu.VMEM((GP, CH), jnp.int32),
        pltpu.VMEM((GP, CH), jnp.int32),
        pltpu.VMEM((GP, CH), jnp.float32),
        pltpu.VMEM((NPT,), jnp.float32),
        pltpu.VMEM_SHARED((NP,), jnp.float32),
        pltpu.SemaphoreType.DMA,
    ],
)


def _edge_body(scaled_hbm, dfix_hbm, src_hbm, agg_out,
               dfix_h, src_h, rows0, rows1, acc_sh,
               g0, g1, s0, s1):
    cid = lax.axis_index("c")
    sid = lax.axis_index("s")
    wid = cid * NS + sid
    base_row = wid * GE
    rows = (rows0, rows1)
    gs = (g0, g1)
    ss = (s0, s1)

    def _zrow(i, carry):
        def _zcol(j, c2):
            rows0[i, pl.ds(j * 16, 16)] = jnp.zeros((16,), jnp.float32)
            return c2

        lax.fori_loop(0, D // 16, _zcol, 0)
        return carry

    lax.fori_loop(0, CHE, _zrow, 0)

    def _zacc(k, carry):
        pltpu.sync_copy(rows0, acc_sh.at[pl.ds(sid * NPT + k * CHE, CHE)])
        return carry

    lax.fori_loop(0, NPT // CHE, _zacc, 0)
    plsc.subcore_barrier()

    def _w_gather(r):
        pltpu.make_async_copy(scaled_hbm.at[dfix_h.at[0]], rows[r],
                              gs[r]).wait()

    def _w_scatter(r):
        pltpu.make_async_copy(rows[r], acc_sh.at[src_h.at[0]], ss[r]).wait()

    def _phase(p, carry):
        row = base_row + p * GPE
        pltpu.sync_copy(dfix_hbm.at[pl.ds(row, GPE)], dfix_h)
        pltpu.sync_copy(src_hbm.at[pl.ds(row, GPE)], src_h)

        for r in range(RE):
            pltpu.async_copy(scaled_hbm.at[dfix_h.at[r]], rows[r], gs[r])
            if r >= 1:
                _w_gather(r - 1)
                pltpu.async_copy(rows[r - 1], acc_sh.at[src_h.at[r - 1]],
                                 ss[r - 1], add=True)

        def _steady(kk, c1):
            for r in range(RE):
                g = kk * RE + r
                _w_scatter(r)
                pltpu.async_copy(scaled_hbm.at[dfix_h.at[g]], rows[r], gs[r])
                q = (r - 1) % RE
                _w_gather(q)
                pltpu.async_copy(rows[q], acc_sh.at[src_h.at[g - 1]],
                                 ss[q], add=True)
            return c1

        lax.fori_loop(1, GPE // RE, _steady, 0)

        _w_gather(RE - 1)
        pltpu.async_copy(rows[RE - 1], acc_sh.at[src_h.at[GPE - 1]],
                         ss[RE - 1], add=True)
        for r in range(RE):
            _w_scatter(r)
        return carry

    lax.fori_loop(0, NPHE, _phase, 0)
    plsc.subcore_barrier()

    def _dump(k, carry):
        sl = pl.ds(sid * NPT + k * CHE, CHE)
        pltpu.sync_copy(acc_sh.at[sl], rows0)
        pltpu.sync_copy(rows0, agg_out.at[cid, sl])
        return carry

    lax.fori_loop(0, NPT // CHE, _dump, 0)


_edge = pl.kernel(
    _edge_body,
    out_type=jax.ShapeDtypeStruct((NC, NP, D), jnp.float32),
    mesh=_MESH,
    scratch_types=[
        pltpu.VMEM((GPE, CHE), jnp.int32),
        pltpu.VMEM((GPE, CHE), jnp.int32),
        pltpu.VMEM((CHE, D), jnp.float32),
        pltpu.VMEM((CHE, D), jnp.float32),
        pltpu.VMEM_SHARED((NP, D), jnp.float32),
        pltpu.SemaphoreType.DMA,
        pltpu.SemaphoreType.DMA,
        pltpu.SemaphoreType.DMA,
        pltpu.SemaphoreType.DMA,
    ],
)

BM = 1024


def _mm_body(x_ref, w_ref, d0_ref, d1_ref, scaled_ref, dinv_ref):
    deg = 1.0 + d0_ref[...] + d1_ref[...]
    dinv = lax.rsqrt(deg)
    acc = jnp.dot(x_ref[...], w_ref[...], preferred_element_type=jnp.float32)
    scaled_ref[...] = acc * dinv[:, None]
    dinv_ref[...] = dinv


_mm = pl.pallas_call(
    _mm_body,
    grid=(pl.cdiv(NSC, BM),),
    in_specs=[
        pl.BlockSpec((BM, D), lambda i: (i, 0)),
        pl.BlockSpec((D, D), lambda i: (0, 0)),
        pl.BlockSpec((BM,), lambda i: (i,)),
        pl.BlockSpec((BM,), lambda i: (i,)),
    ],
    out_specs=[
        pl.BlockSpec((BM, D), lambda i: (i, 0)),
        pl.BlockSpec((BM,), lambda i: (i,)),
    ],
    out_shape=[
        jax.ShapeDtypeStruct((NSC, D), jnp.float32),
        jax.ShapeDtypeStruct((NSC,), jnp.float32),
    ],
)


def _comb_body(agg_ref0, agg_ref1, s_ref, dinv_ref, b_ref, o_ref):
    total = agg_ref0[0] + agg_ref1[0] + s_ref[...]
    o_ref[...] = total * dinv_ref[...][:, None] + b_ref[...]


_comb = pl.pallas_call(
    _comb_body,
    grid=(pl.cdiv(N, BM),),
    in_specs=[
        pl.BlockSpec((1, BM, D), lambda i: (0, i, 0)),
        pl.BlockSpec((1, BM, D), lambda i: (1, i, 0)),
        pl.BlockSpec((BM, D), lambda i: (i, 0)),
        pl.BlockSpec((BM,), lambda i: (i,)),
        pl.BlockSpec((1, D), lambda i: (0, 0)),
    ],
    out_specs=pl.BlockSpec((BM, D), lambda i: (i, 0)),
    out_shape=jax.ShapeDtypeStruct((N, D), jnp.float32),
)


def kernel(input, adj, weight, bias):
    src = adj[0]
    dst = adj[1]
    pad = EPAD - E
    padv = jnp.arange(pad, dtype=jnp.int32)
    src2d = jnp.concatenate([src, padv]).reshape(TOTR, CH)
    dst2d = jnp.concatenate([dst, padv]).reshape(TOTR, CH)
    x_p = jnp.pad(input, ((0, NSC - N), (0, 0)))

    degp, dfix2d = _deg(src2d, dst2d)
    d0 = jnp.pad(degp[0], (0, NSC - NP))
    d1 = jnp.pad(degp[1], (0, NSC - NP))
    scaled, dinv = _mm(x_p, weight, d0, d1)
    aggp = _edge(scaled, dfix2d, src2d)
    return _comb(aggp, aggp, scaled, dinv, bias.reshape(1, D))


# speedup vs baseline: 1.1572x; 1.0374x over previous
import jax
import jax.numpy as jnp
from jax import lax
from jax.experimental import pallas as pl
from jax.experimental.pallas import tpu as pltpu
from jax.experimental.pallas import tpu_sc as plsc

N = 10000
D = 128
E = 320000

NC = 2
NS = 16
NT = NC * NS

NP = 10240
NZ = 2048
NSC = N + NZ
Z = N
NPT = NP // NS

CH = 128
GP = 40
G = 80
NPH = G // GP
TOTR = NT * G
EPAD = TOTR * CH

CHE = 128
GE = 80
GPE = 40
NPHE = GE // GPE
RE = 2

_MESH = plsc.VectorSubcoreMesh(
    core_axis_name="c", subcore_axis_name="s", num_cores=NC, num_subcores=NS
)


def _deg_body(src_hbm, dst_hbm, degp_out, dfix_out,
              src_all, dst_all, dfix_all, val_all, zer_v, deg_sh, sem):
    cid = lax.axis_index("c")
    sid = lax.axis_index("s")
    wid = cid * NS + sid
    base_row = wid * G

    def _fill_zeros(i, carry):
        zer_v[pl.ds(i * 16, 16)] = jnp.zeros((16,), jnp.float32)
        return carry

    lax.fori_loop(0, NPT // 16, _fill_zeros, 0)
    pltpu.sync_copy(zer_v, deg_sh.at[pl.ds(sid * NPT, NPT)])
    plsc.subcore_barrier()

    def _phase(p, carry):
        row = base_row + p * GP
        pltpu.sync_copy(src_hbm.at[pl.ds(row, GP)], src_all)
        pltpu.sync_copy(dst_hbm.at[pl.ds(row, GP)], dst_all)

        def _fix_row(g, c1):
            def _fix(j, c2):
                s = src_all[g, pl.ds(j * 16, 16)]
                d = dst_all[g, pl.ds(j * 16, 16)]
                diag = s == d
                lane = lax.iota(jnp.int32, 16)
                zidx = Z + (((row + g) * CH + j * 16 + lane) & (NZ - 1))
                dfix_all[g, pl.ds(j * 16, 16)] = jnp.where(diag, zidx, d)
                val_all[g, pl.ds(j * 16, 16)] = jnp.where(
                    diag, jnp.zeros((16,), jnp.float32),
                    jnp.ones((16,), jnp.float32))
                return c2

            lax.fori_loop(0, CH // 16, _fix, 0)
            return c1

        lax.fori_loop(0, GP, _fix_row, 0)
        pltpu.sync_copy(dfix_all, dfix_out.at[pl.ds(row, GP)])

        def _wave(kk, c1):
            for r in range(8):
                g = kk * 8 + r
                pltpu.async_copy(val_all.at[g], deg_sh.at[src_all.at[g]],
                                 sem, add=True)
            for r in range(8):
                pltpu.make_async_copy(val_all.at[0],
                                      deg_sh.at[src_all.at[0]], sem).wait()
            return c1

        lax.fori_loop(0, GP // 8, _wave, 0)
        return carry

    lax.fori_loop(0, NPH, _phase, 0)
    plsc.subcore_barrier()
    pltpu.sync_copy(deg_sh.at[pl.ds(sid * NPT, NPT)], zer_v)
    pltpu.sync_copy(zer_v, degp_out.at[cid, pl.ds(sid * NPT, NPT)])


_deg = pl.kernel(
    _deg_body,
    out_type=(
        jax.ShapeDtypeStruct((NC, NP), jnp.float32),
        jax.ShapeDtypeStruct((TOTR, CH), jnp.int32),
    ),
    mesh=_MESH,
    scratch_types=[
        pltpu.VMEM((GP, CH), jnp.int32),
        pltpu.VMEM((GP, CH), jnp.int32),
        pltpu.VMEM((GP, CH), jnp.int32),
        pltpu.VMEM((GP, CH), jnp.float32),
        pltpu.VMEM((NPT,), jnp.float32),
        pltpu.VMEM_SHARED((NP,), jnp.float32),
        pltpu.SemaphoreType.DMA,
    ],
)


def _edge_body(scaled_hbm, dfix_hbm, src_hbm, agg_out,
               dfix_h, src_h, rows0, rows1, acc_sh,
               g0, g1, s0, s1):
    cid = lax.axis_index("c")
    sid = lax.axis_index("s")
    wid = cid * NS + sid
    base_row = wid * GE
    rows = (rows0, rows1)
    gs = (g0, g1)
    ss = (s0, s1)

    def _zrow(i, carry):
        def _zcol(j, c2):
            rows0[i, pl.ds(j * 16, 16)] = jnp.zeros((16,), jnp.float32)
            return c2

        lax.fori_loop(0, D // 16, _zcol, 0)
        return carry

    lax.fori_loop(0, CHE, _zrow, 0)

    def _zacc(k, carry):
        pltpu.sync_copy(rows0, acc_sh.at[pl.ds(sid * NPT + k * CHE, CHE)])
        return carry

    lax.fori_loop(0, NPT // CHE, _zacc, 0)
    plsc.subcore_barrier()

    def _w_gather(r):
        pltpu.make_async_copy(scaled_hbm.at[dfix_h.at[0]], rows[r],
                              gs[r]).wait()

    def _w_scatter(r):
        pltpu.make_async_copy(rows[r], acc_sh.at[src_h.at[0]], ss[r]).wait()

    def _phase(p, carry):
        row = base_row + p * GPE
        pltpu.sync_copy(dfix_hbm.at[pl.ds(row, GPE)], dfix_h)
        pltpu.sync_copy(src_hbm.at[pl.ds(row, GPE)], src_h)

        for r in range(RE):
            pltpu.async_copy(scaled_hbm.at[dfix_h.at[r]], rows[r], gs[r])
            if r >= 1:
                _w_gather(r - 1)
                pltpu.async_copy(rows[r - 1], acc_sh.at[src_h.at[r - 1]],
                                 ss[r - 1], add=True)

        def _steady(kk, c1):
            for r in range(RE):
                g = kk * RE + r
                _w_scatter(r)
                pltpu.async_copy(scaled_hbm.at[dfix_h.at[g]], rows[r], gs[r])
                q = (r - 1) % RE
                _w_gather(q)
                pltpu.async_copy(rows[q], acc_sh.at[src_h.at[g - 1]],
                                 ss[q], add=True)
            return c1

        lax.fori_loop(1, GPE // RE, _steady, 0)

        _w_gather(RE - 1)
        pltpu.async_copy(rows[RE - 1], acc_sh.at[src_h.at[GPE - 1]],
                         ss[RE - 1], add=True)
        for r in range(RE):
            _w_scatter(r)
        return carry

    lax.fori_loop(0, NPHE, _phase, 0)
    plsc.subcore_barrier()

    def _dump(k, carry):
        sl = pl.ds(sid * NPT + k * CHE, CHE)
        pltpu.sync_copy(acc_sh.at[sl], rows0)
        pltpu.sync_copy(rows0, agg_out.at[cid, sl])
        return carry

    lax.fori_loop(0, NPT // CHE, _dump, 0)


_edge = pl.kernel(
    _edge_body,
    out_type=jax.ShapeDtypeStruct((NC, NP, D), jnp.float32),
    mesh=_MESH,
    scratch_types=[
        pltpu.VMEM((GPE, CHE), jnp.int32),
        pltpu.VMEM((GPE, CHE), jnp.int32),
        pltpu.VMEM((CHE, D), jnp.float32),
        pltpu.VMEM((CHE, D), jnp.float32),
        pltpu.VMEM_SHARED((NP, D), jnp.float32),
        pltpu.SemaphoreType.DMA,
        pltpu.SemaphoreType.DMA,
        pltpu.SemaphoreType.DMA,
        pltpu.SemaphoreType.DMA,
    ],
)

PADE = EPAD - E
PADR = PADE // CH


def _pack_body(adj_ref, src_ref, dst_ref):
    flat = adj_ref[...]
    a2 = flat[:E].reshape(E // CH, CH)
    b2 = flat[E:].reshape(E // CH, CH)
    padv = (lax.broadcasted_iota(jnp.int32, (PADR, CH), 0) * CH
            + lax.broadcasted_iota(jnp.int32, (PADR, CH), 1))
    src_ref[...] = jnp.concatenate([a2, padv], axis=0)
    dst_ref[...] = jnp.concatenate([b2, padv], axis=0)


_pack = pl.pallas_call(
    _pack_body,
    out_shape=[
        jax.ShapeDtypeStruct((TOTR, CH), jnp.int32),
        jax.ShapeDtypeStruct((TOTR, CH), jnp.int32),
    ],
)

BM = 1024


def _mm_body(x_ref, w_ref, d0_ref, d1_ref, scaled_ref, dinv_ref):
    deg = 1.0 + d0_ref[...] + d1_ref[...]
    dinv = lax.rsqrt(deg)
    acc = jnp.dot(x_ref[...], w_ref[...], preferred_element_type=jnp.float32)
    scaled_ref[...] = acc * dinv[:, None]
    dinv_ref[...] = dinv


_mm = pl.pallas_call(
    _mm_body,
    grid=(pl.cdiv(NSC, BM),),
    in_specs=[
        pl.BlockSpec((BM, D), lambda i: (i, 0)),
        pl.BlockSpec((D, D), lambda i: (0, 0)),
        pl.BlockSpec((BM,), lambda i: (i,)),
        pl.BlockSpec((BM,), lambda i: (i,)),
    ],
    out_specs=[
        pl.BlockSpec((BM, D), lambda i: (i, 0)),
        pl.BlockSpec((BM,), lambda i: (i,)),
    ],
    out_shape=[
        jax.ShapeDtypeStruct((NSC, D), jnp.float32),
        jax.ShapeDtypeStruct((NSC,), jnp.float32),
    ],
)


def _comb_body(agg_ref0, agg_ref1, s_ref, dinv_ref, b_ref, o_ref):
    total = agg_ref0[0] + agg_ref1[0] + s_ref[...]
    o_ref[...] = total * dinv_ref[...][:, None] + b_ref[...]


_comb = pl.pallas_call(
    _comb_body,
    grid=(pl.cdiv(N, BM),),
    in_specs=[
        pl.BlockSpec((1, BM, D), lambda i: (0, i, 0)),
        pl.BlockSpec((1, BM, D), lambda i: (1, i, 0)),
        pl.BlockSpec((BM, D), lambda i: (i, 0)),
        pl.BlockSpec((BM,), lambda i: (i,)),
        pl.BlockSpec((1, D), lambda i: (0, 0)),
    ],
    out_specs=pl.BlockSpec((BM, D), lambda i: (i, 0)),
    out_shape=jax.ShapeDtypeStruct((N, D), jnp.float32),
)


def kernel(input, adj, weight, bias):
    src2d, dst2d = _pack(adj.reshape(2 * E))
    x_p = jnp.pad(input, ((0, NSC - N), (0, 0)))

    degp, dfix2d = _deg(src2d, dst2d)
    d0 = jnp.pad(degp[0], (0, NSC - NP))
    d1 = jnp.pad(degp[1], (0, NSC - NP))
    scaled, dinv = _mm(x_p, weight, d0, d1)
    aggp = _edge(scaled, dfix2d, src2d)
    return _comb(aggp, aggp, scaled, dinv, bias.reshape(1, D))


# speedup vs baseline: 1.1896x; 1.0280x over previous
import jax
import jax.numpy as jnp
from jax import lax
from jax.experimental import pallas as pl
from jax.experimental.pallas import tpu as pltpu
from jax.experimental.pallas import tpu_sc as plsc

N = 10000
D = 128
E = 320000

NC = 2
NS = 16
NT = NC * NS

NP = 10240
NZ = 2048
NSC = N + NZ
Z = N
NPT = NP // NS

CH = 128
GP = 40
G = 80
NPH = G // GP
TOTR = NT * G
EPAD = TOTR * CH

CHE = 128
GE = 80
GPE = 40
NPHE = GE // GPE
RE = 2

_MESH = plsc.VectorSubcoreMesh(
    core_axis_name="c", subcore_axis_name="s", num_cores=NC, num_subcores=NS
)


def _deg_body(src_hbm, dst_hbm, degp_out, dfix_out,
              src_all, dst_all, dfix_all, val_all, zer_v, deg_sh, sem):
    cid = lax.axis_index("c")
    sid = lax.axis_index("s")
    wid = cid * NS + sid
    base_row = wid * G

    def _fill_zeros(i, carry):
        zer_v[pl.ds(i * 16, 16)] = jnp.zeros((16,), jnp.float32)
        return carry

    lax.fori_loop(0, NPT // 16, _fill_zeros, 0)
    pltpu.sync_copy(zer_v, deg_sh.at[pl.ds(sid * NPT, NPT)])
    plsc.subcore_barrier()

    def _phase(p, carry):
        row = base_row + p * GP
        pltpu.sync_copy(src_hbm.at[pl.ds(row, GP)], src_all)
        pltpu.sync_copy(dst_hbm.at[pl.ds(row, GP)], dst_all)

        def _fix_row(g, c1):
            def _fix(j, c2):
                s = src_all[g, pl.ds(j * 16, 16)]
                d = dst_all[g, pl.ds(j * 16, 16)]
                diag = s == d
                lane = lax.iota(jnp.int32, 16)
                zidx = Z + (((row + g) * CH + j * 16 + lane) & (NZ - 1))
                dfix_all[g, pl.ds(j * 16, 16)] = jnp.where(diag, zidx, d)
                val_all[g, pl.ds(j * 16, 16)] = jnp.where(
                    diag, jnp.zeros((16,), jnp.float32),
                    jnp.ones((16,), jnp.float32))
                return c2

            lax.fori_loop(0, CH // 16, _fix, 0)
            return c1

        lax.fori_loop(0, GP, _fix_row, 0)
        pltpu.sync_copy(dfix_all, dfix_out.at[pl.ds(row, GP)])

        def _wave(kk, c1):
            for r in range(8):
                g = kk * 8 + r
                pltpu.async_copy(val_all.at[g], deg_sh.at[src_all.at[g]],
                                 sem, add=True)
            for r in range(8):
                pltpu.make_async_copy(val_all.at[0],
                                      deg_sh.at[src_all.at[0]], sem).wait()
            return c1

        lax.fori_loop(0, GP // 8, _wave, 0)
        return carry

    lax.fori_loop(0, NPH, _phase, 0)
    plsc.subcore_barrier()
    pltpu.sync_copy(deg_sh.at[pl.ds(sid * NPT, NPT)], zer_v)
    pltpu.sync_copy(zer_v, degp_out.at[cid, pl.ds(sid * NPT, NPT)])


_deg = pl.kernel(
    _deg_body,
    out_type=(
        jax.ShapeDtypeStruct((NC, NP), jnp.float32),
        jax.ShapeDtypeStruct((TOTR, CH), jnp.int32),
    ),
    mesh=_MESH,
    scratch_types=[
        pltpu.VMEM((GP, CH), jnp.int32),
        pltpu.VMEM((GP, CH), jnp.int32),
        pltpu.VMEM((GP, CH), jnp.int32),
        pltpu.VMEM((GP, CH), jnp.float32),
        pltpu.VMEM((NPT,), jnp.float32),
        pltpu.VMEM_SHARED((NP,), jnp.float32),
        pltpu.SemaphoreType.DMA,
    ],
)


def _edge_body(scaled_hbm, dfix_hbm, src_hbm, agg_out,
               dfix_h, src_h, rows0, rows1, acc_sh,
               g0, g1, s0, s1):
    cid = lax.axis_index("c")
    sid = lax.axis_index("s")
    wid = cid * NS + sid
    base_row = wid * GE
    rows = (rows0, rows1)
    gs = (g0, g1)
    ss = (s0, s1)

    def _zrow(i, carry):
        def _zcol(j, c2):
            rows0[i, pl.ds(j * 16, 16)] = jnp.zeros((16,), jnp.float32)
            return c2

        lax.fori_loop(0, D // 16, _zcol, 0)
        return carry

    lax.fori_loop(0, CHE, _zrow, 0)

    def _zacc(k, carry):
        pltpu.sync_copy(rows0, acc_sh.at[pl.ds(sid * NPT + k * CHE, CHE)])
        return carry

    lax.fori_loop(0, NPT // CHE, _zacc, 0)
    plsc.subcore_barrier()

    def _w_gather(r):
        pltpu.make_async_copy(scaled_hbm.at[dfix_h.at[0]], rows[r],
                              gs[r]).wait()

    def _w_scatter(r):
        pltpu.make_async_copy(rows[r], acc_sh.at[src_h.at[0]], ss[r]).wait()

    def _phase(p, carry):
        row = base_row + p * GPE
        pltpu.sync_copy(dfix_hbm.at[pl.ds(row, GPE)], dfix_h)
        pltpu.sync_copy(src_hbm.at[pl.ds(row, GPE)], src_h)

        for r in range(RE):
            pltpu.async_copy(scaled_hbm.at[dfix_h.at[r]], rows[r], gs[r])
            if r >= 1:
                _w_gather(r - 1)
                pltpu.async_copy(rows[r - 1], acc_sh.at[src_h.at[r - 1]],
                                 ss[r - 1], add=True)

        def _steady(kk, c1):
            for r in range(RE):
                g = kk * RE + r
                _w_scatter(r)
                pltpu.async_copy(scaled_hbm.at[dfix_h.at[g]], rows[r], gs[r])
                q = (r - 1) % RE
                _w_gather(q)
                pltpu.async_copy(rows[q], acc_sh.at[src_h.at[g - 1]],
                                 ss[q], add=True)
            return c1

        lax.fori_loop(1, GPE // RE, _steady, 0)

        _w_gather(RE - 1)
        pltpu.async_copy(rows[RE - 1], acc_sh.at[src_h.at[GPE - 1]],
                         ss[RE - 1], add=True)
        for r in range(RE):
            _w_scatter(r)
        return carry

    lax.fori_loop(0, NPHE, _phase, 0)
    plsc.subcore_barrier()

    def _dump(k, carry):
        sl = pl.ds(sid * NPT + k * CHE, CHE)
        pltpu.sync_copy(acc_sh.at[sl], rows0)
        pltpu.sync_copy(rows0, agg_out.at[cid, sl])
        return carry

    lax.fori_loop(0, NPT // CHE, _dump, 0)


_edge = pl.kernel(
    _edge_body,
    out_type=jax.ShapeDtypeStruct((NC, NP, D), jnp.float32),
    mesh=_MESH,
    scratch_types=[
        pltpu.VMEM((GPE, CHE), jnp.int32),
        pltpu.VMEM((GPE, CHE), jnp.int32),
        pltpu.VMEM((CHE, D), jnp.float32),
        pltpu.VMEM((CHE, D), jnp.float32),
        pltpu.VMEM_SHARED((NP, D), jnp.float32),
        pltpu.SemaphoreType.DMA,
        pltpu.SemaphoreType.DMA,
        pltpu.SemaphoreType.DMA,
        pltpu.SemaphoreType.DMA,
    ],
)

PADE = EPAD - E
PADR = PADE // CH


def _pack_body(adj_ref, src_ref, dst_ref):
    a2 = adj_ref[0].reshape(E // CH, CH)
    b2 = adj_ref[1].reshape(E // CH, CH)
    padv = (lax.broadcasted_iota(jnp.int32, (PADR, CH), 0) * CH
            + lax.broadcasted_iota(jnp.int32, (PADR, CH), 1))
    src_ref[...] = jnp.concatenate([a2, padv], axis=0)
    dst_ref[...] = jnp.concatenate([b2, padv], axis=0)


_pack = pl.pallas_call(
    _pack_body,
    out_shape=[
        jax.ShapeDtypeStruct((TOTR, CH), jnp.int32),
        jax.ShapeDtypeStruct((TOTR, CH), jnp.int32),
    ],
)

BM = 1024


def _mm_body(x_ref, w_ref, d0_ref, d1_ref, scaled_ref, dinv_ref):
    deg = 1.0 + d0_ref[...] + d1_ref[...]
    dinv = lax.rsqrt(deg)
    acc = jnp.dot(x_ref[...], w_ref[...], preferred_element_type=jnp.float32)
    scaled_ref[...] = acc * dinv[:, None]
    dinv_ref[...] = dinv


_mm = pl.pallas_call(
    _mm_body,
    grid=(pl.cdiv(NSC, BM),),
    in_specs=[
        pl.BlockSpec((BM, D), lambda i: (i, 0)),
        pl.BlockSpec((D, D), lambda i: (0, 0)),
        pl.BlockSpec((BM,), lambda i: (i,)),
        pl.BlockSpec((BM,), lambda i: (i,)),
    ],
    out_specs=[
        pl.BlockSpec((BM, D), lambda i: (i, 0)),
        pl.BlockSpec((BM,), lambda i: (i,)),
    ],
    out_shape=[
        jax.ShapeDtypeStruct((NSC, D), jnp.float32),
        jax.ShapeDtypeStruct((NSC,), jnp.float32),
    ],
)


def _comb_body(agg_ref0, agg_ref1, s_ref, dinv_ref, b_ref, o_ref):
    total = agg_ref0[0] + agg_ref1[0] + s_ref[...]
    o_ref[...] = total * dinv_ref[...][:, None] + b_ref[...]


_comb = pl.pallas_call(
    _comb_body,
    grid=(pl.cdiv(N, BM),),
    in_specs=[
        pl.BlockSpec((1, BM, D), lambda i: (0, i, 0)),
        pl.BlockSpec((1, BM, D), lambda i: (1, i, 0)),
        pl.BlockSpec((BM, D), lambda i: (i, 0)),
        pl.BlockSpec((BM,), lambda i: (i,)),
        pl.BlockSpec((1, D), lambda i: (0, 0)),
    ],
    out_specs=pl.BlockSpec((BM, D), lambda i: (i, 0)),
    out_shape=jax.ShapeDtypeStruct((N, D), jnp.float32),
)


def kernel(input, adj, weight, bias):
    src2d, dst2d = _pack(adj)
    x_p = jnp.pad(input, ((0, NSC - N), (0, 0)))

    degp, dfix2d = _deg(src2d, dst2d)
    d0 = jnp.pad(degp[0], (0, NSC - NP))
    d1 = jnp.pad(degp[1], (0, NSC - NP))
    scaled, dinv = _mm(x_p, weight, d0, d1)
    aggp = _edge(scaled, dfix2d, src2d)
    return _comb(aggp, aggp, scaled, dinv, bias.reshape(1, D))
